# Initial kernel scaffold; baseline (speedup 1.0000x reference)
import jax
import jax.numpy as jnp
from jax.experimental import pallas as pl


def kernel(begin, end, forward, backward):
    raise NotImplementedError("write your pallas kernel here")


# speedup vs baseline: 1.4547x; 1.4547x over previous
import functools

import jax
import jax.numpy as jnp
from jax import lax
from jax.experimental import pallas as pl
from jax.experimental.pallas import tpu as pltpu
from jax.experimental.pallas import tpu_sc as plsc


def _build(B, S, D):
  info = plsc.get_sparse_core_info()
  NC, NS, L = info.num_cores, info.num_subcores, info.num_lanes
  NW = NC * NS
  assert B % (8 * NW) == 0, "batch must split 8-aligned across subcores"
  bpw = B // NW

  mesh = plsc.VectorSubcoreMesh(core_axis_name="c", subcore_axis_name="s")

  @functools.partial(
      pl.kernel,
      mesh=mesh,
      out_type=jax.ShapeDtypeStruct((B, 2 * D), jnp.float32),
      scratch_types=[
          pltpu.VMEM((bpw,), jnp.int32),
          pltpu.VMEM((bpw,), jnp.int32),
          pltpu.VMEM((bpw, D), jnp.float32),
          pltpu.VMEM((bpw, D), jnp.float32),
          pltpu.SemaphoreType.DMA,
      ],
  )
  def k(begin_hbm, end_hbm, fwd_hbm, bwd_hbm, out_hbm,
        bidx, eidx, brows, erows, sem):
    wid = lax.axis_index("s") * NC + lax.axis_index("c")
    base = wid * bpw
    pltpu.sync_copy(begin_hbm.at[pl.ds(base, bpw)], bidx)
    pltpu.sync_copy(end_hbm.at[pl.ds(base, bpw)], eidx)
    lane = lax.iota(jnp.int32, L)
    for j in range(bpw // L):
      sl = pl.ds(j * L, L)
      off = (base + j * L + lane) * S
      bidx[sl] = bidx[sl] + off
      eidx[sl] = eidx[sl] + off
    cb = pltpu.async_copy(bwd_hbm.at[bidx], brows, sem)
    ce = pltpu.async_copy(fwd_hbm.at[eidx], erows, sem)
    cb.wait()
    ce.wait()
    pltpu.sync_copy(brows, out_hbm.at[pl.ds(base, bpw), pl.ds(0, D)])
    pltpu.sync_copy(erows, out_hbm.at[pl.ds(base, bpw), pl.ds(D, D)])

  return k


def kernel(begin, end, forward, backward):
  B, S, D = forward.shape
  begin_f = begin.reshape(B).astype(jnp.int32)
  end_f = end.reshape(B).astype(jnp.int32)
  fwd = forward.reshape(B * S, D)
  bwd = backward.reshape(B * S, D)
  return _build(B, S, D)(begin_f, end_f, fwd, bwd)
